# baseline (speedup 1.0000x reference)
import jax
import jax.numpy as jnp
from jax.experimental import pallas as pl
from jax.experimental.pallas import tpu as pltpu

F32 = jnp.float32
B, N = 32, 64
NB = 3 * B
D_CAT = N * 192
D_H = 4096
D_OUT = 2048
FC1_BLK = 128
FC2_BLK = 128
NBUF = 4
FC1_CHUNKS = D_H // FC1_BLK
FC2_CHUNKS = D_OUT // FC2_BLK
LAG = NBUF
FC2_START = FC1_CHUNKS + LAG
NSTEPS = FC2_START + FC2_CHUNKS


def _fused_kernel(x0_ref, x1_ref, x2_ref,
                  aon0_ref, aon1_ref, aon2_ref,
                  ain0_ref, ain1_ref, ain2_ref,
                  apr0_ref, apr1_ref, apr2_ref,
                  won0_ref, bon0_ref, won1_ref, bon1_ref,
                  win0_ref, bin0_ref, win1_ref, bin1_ref,
                  wpr0_ref, bpr0_ref, wpr1_ref, bpr1_ref,
                  fc1w_hbm, fc1b_ref, fc2w_ref, fc2b_ref,
                  out_ref, cat3_scr, cat2_scr, h_scr, w_buf,
                  x_scr, a_scr, w_sem, x_sem, a_sem):
    step = pl.program_id(0)
    xs_hbm = (x0_ref, x1_ref, x2_ref)
    as_hbm = (aon0_ref, aon1_ref, aon2_ref,
              ain0_ref, ain1_ref, ain2_ref,
              apr0_ref, apr1_ref, apr2_ref)

    def w_copy(c, nb):
        return pltpu.make_async_copy(
            fc1w_hbm.at[:, pl.ds(c * FC1_BLK, FC1_BLK)],
            w_buf.at[nb], w_sem.at[nb])


    def x_copy(m):
        return pltpu.make_async_copy(xs_hbm[m], x_scr.at[m], x_sem.at[m])

    def a_copy(m, p):
        return pltpu.make_async_copy(
            as_hbm[3 * p + m], a_scr.at[3 * p + m], a_sem.at[3 * p + m])

    @pl.when(step == 0)
    def _kick_first():
        x_copy(0).start()
        for p in range(3):
            a_copy(0, p).start()
        w_copy(0, 0).start()
        for m in (1, 2):
            x_copy(m).start()
            for p in range(3):
                a_copy(m, p).start()
        for c in range(1, NBUF):
            w_copy(c, c).start()

    @pl.when(step < 3)
    def _gcn():
        paths = (
            (won0_ref, bon0_ref, won1_ref, bon1_ref, 0),
            (win0_ref, bin0_ref, win1_ref, bin1_ref, 64),
            (wpr0_ref, bpr0_ref, wpr1_ref, bpr1_ref, 128),
        )
        for m in range(3):
            @pl.when(step == m)
            def _member():
                x_copy(m).wait()
                x2 = x_scr[m].reshape(B * N, 256)
                for p, (w0_ref, b0_ref, w1_ref, b1_ref, off) in enumerate(paths):
                    a_copy(m, p).wait()
                    a = a_scr[3 * p + m]
                    s0 = jnp.dot(x2, w0_ref[...], preferred_element_type=F32)
                    h0 = jax.lax.dot_general(
                        a, s0.reshape(B, N, 128), (((2,), (1,)), ((0,), (0,))),
                        preferred_element_type=F32)
                    h0 = jnp.maximum(h0 + b0_ref[...], 0.0)
                    s1 = jnp.dot(h0.reshape(B * N, 128), w1_ref[...],
                                 preferred_element_type=F32)
                    h1 = jax.lax.dot_general(
                        a, s1.reshape(B, N, 64), (((2,), (1,)), ((0,), (0,))),
                        preferred_element_type=F32)
                    cat3_scr[m * B:(m + 1) * B, :, off:off + 64] = \
                        jnp.maximum(h1 + b1_ref[...], 0.0)
                cat2_scr[m * B:(m + 1) * B, :] = \
                    cat3_scr[m * B:(m + 1) * B].reshape(B, D_CAT)

    @pl.when((step >= LAG) & (step < FC2_START))
    def _fc1():
        c = step - LAG
        nb = jax.lax.rem(c, NBUF)
        w_copy(c, nb).wait()
        blk = jnp.dot(cat2_scr[...], w_buf[nb], preferred_element_type=F32)
        h_scr[:, pl.ds(c * FC1_BLK, FC1_BLK)] = \
            jnp.maximum(blk + fc1b_ref[...], 0.0)

    @pl.when((step >= LAG) & (step < FC1_CHUNKS))
    def _kick_next():
        w_copy(step, jax.lax.rem(step, NBUF)).start()

    @pl.when(step >= FC2_START)
    def _fc2():
        blk = jnp.dot(h_scr[...], fc2w_ref[...], preferred_element_type=F32)
        out_ref[...] = jnp.maximum(blk + fc2b_ref[...], 0.0)


def kernel(A_on, P_on, N_on, A_in, P_in, N_in, A_prox, P_prox, N_prox,
           A_X, P_X, N_X,
           W_on_0, b_on_0, W_on_1, b_on_1,
           W_in_0, b_in_0, W_in_1, b_in_1,
           W_prox_0, b_prox_0, W_prox_1, b_prox_1,
           fc1_W, fc1_b, fc2_W, fc2_b):
    r = lambda b: b.reshape(1, -1)
    const = lambda arr: pl.BlockSpec(arr.shape, lambda i: (0,) * arr.ndim)
    hbm = lambda: pl.BlockSpec(memory_space=pltpu.MemorySpace.HBM)
    data_inputs = (A_X, P_X, N_X, A_on, P_on, N_on, A_in, P_in, N_in,
                   A_prox, P_prox, N_prox)
    weight_inputs = (W_on_0, r(b_on_0), W_on_1, r(b_on_1),
                     W_in_0, r(b_in_0), W_in_1, r(b_in_1),
                     W_prox_0, r(b_prox_0), W_prox_1, r(b_prox_1))
    in_specs = [hbm() for _ in data_inputs]
    in_specs += [const(a) for a in weight_inputs]
    in_specs += [
        hbm(),
        pl.BlockSpec((1, FC1_BLK),
                     lambda i: (0, jnp.clip(i - LAG, 0, FC1_CHUNKS - 1))),
        pl.BlockSpec((D_H, FC2_BLK),
                     lambda i: (0, jnp.maximum(i - FC2_START, 0))),
        pl.BlockSpec((1, FC2_BLK),
                     lambda i: (0, jnp.maximum(i - FC2_START, 0))),
    ]
    out = pl.pallas_call(
        _fused_kernel,
        grid=(NSTEPS,),
        in_specs=in_specs,
        out_specs=pl.BlockSpec((NB, FC2_BLK),
                               lambda i: (0, jnp.maximum(i - FC2_START, 0))),
        out_shape=jax.ShapeDtypeStruct((NB, D_OUT), F32),
        scratch_shapes=[
            pltpu.VMEM((NB, N, 192), F32),
            pltpu.VMEM((NB, D_CAT), F32),
            pltpu.VMEM((NB, D_H), F32),
            pltpu.VMEM((NBUF, D_CAT, FC1_BLK), F32),
            pltpu.VMEM((3, B, N, 256), F32),
            pltpu.VMEM((9, B, N, N), F32),
            pltpu.SemaphoreType.DMA((NBUF,)),
            pltpu.SemaphoreType.DMA((3,)),
            pltpu.SemaphoreType.DMA((9,)),
        ],
    )(*data_inputs, *weight_inputs, fc1_W, r(fc1_b), fc2_W, r(fc2_b))
    return (out[0:B], out[B:2 * B], out[2 * B:3 * B])


# speedup vs baseline: 1.0951x; 1.0951x over previous
import jax
import jax.numpy as jnp
from jax.experimental import pallas as pl
from jax.experimental.pallas import tpu as pltpu

F32 = jnp.float32
B, N = 32, 64
NB = 3 * B
D_CAT = N * 192
D_H = 4096
D_OUT = 2048
FC1_BLK = 128
FC2_BLK = 256
NBUF = 3
FC1_CHUNKS = D_H // FC1_BLK
FC2_CHUNKS = D_OUT // FC2_BLK
LAG = NBUF
FC2_START = FC1_CHUNKS + LAG
NSTEPS = FC2_START + FC2_CHUNKS


def _fused_kernel(x0_ref, x1_ref, x2_ref,
                  aon0_ref, aon1_ref, aon2_ref,
                  ain0_ref, ain1_ref, ain2_ref,
                  apr0_ref, apr1_ref, apr2_ref,
                  won0_ref, bon0_ref, won1_ref, bon1_ref,
                  win0_ref, bin0_ref, win1_ref, bin1_ref,
                  wpr0_ref, bpr0_ref, wpr1_ref, bpr1_ref,
                  fc1w_hbm, fc1b_ref, fc2w_ref, fc2b_ref,
                  out_ref, cat3_scr, cat2_scr, h_scr, w_buf,
                  x_scr, a_scr, w_sem, x_sem, a_sem):
    step = pl.program_id(0)
    xs_hbm = (x0_ref, x1_ref, x2_ref)
    as_hbm = (aon0_ref, aon1_ref, aon2_ref,
              ain0_ref, ain1_ref, ain2_ref,
              apr0_ref, apr1_ref, apr2_ref)

    def w_copy(c, nb):
        return pltpu.make_async_copy(
            fc1w_hbm.at[:, pl.ds(c * FC1_BLK, FC1_BLK)],
            w_buf.at[nb], w_sem.at[nb])


    def x_copy(m):
        return pltpu.make_async_copy(xs_hbm[m], x_scr.at[m], x_sem.at[m])

    def a_copy(m, p):
        return pltpu.make_async_copy(
            as_hbm[3 * p + m], a_scr.at[3 * p + m], a_sem.at[3 * p + m])

    @pl.when(step == 0)
    def _kick_first():
        x_copy(0).start()
        for p in range(3):
            a_copy(0, p).start()
        w_copy(0, 0).start()
        for m in (1, 2):
            x_copy(m).start()
            for p in range(3):
                a_copy(m, p).start()
        for c in range(1, NBUF):
            w_copy(c, c).start()

    @pl.when(step < 3)
    def _gcn():
        paths = (
            (won0_ref, bon0_ref, won1_ref, bon1_ref, 0),
            (win0_ref, bin0_ref, win1_ref, bin1_ref, 64),
            (wpr0_ref, bpr0_ref, wpr1_ref, bpr1_ref, 128),
        )
        for m in range(3):
            @pl.when(step == m)
            def _member():
                x_copy(m).wait()
                x2 = x_scr[m].reshape(B * N, 256)
                for p, (w0_ref, b0_ref, w1_ref, b1_ref, off) in enumerate(paths):
                    a_copy(m, p).wait()
                    a = a_scr[3 * p + m]
                    s0 = jnp.dot(x2, w0_ref[...], preferred_element_type=F32)
                    h0 = jax.lax.dot_general(
                        a, s0.reshape(B, N, 128), (((2,), (1,)), ((0,), (0,))),
                        preferred_element_type=F32)
                    h0 = jnp.maximum(h0 + b0_ref[...], 0.0)
                    s1 = jnp.dot(h0.reshape(B * N, 128), w1_ref[...],
                                 preferred_element_type=F32)
                    h1 = jax.lax.dot_general(
                        a, s1.reshape(B, N, 64), (((2,), (1,)), ((0,), (0,))),
                        preferred_element_type=F32)
                    cat3_scr[m * B:(m + 1) * B, :, off:off + 64] = \
                        jnp.maximum(h1 + b1_ref[...], 0.0)
                cat2_scr[m * B:(m + 1) * B, :] = \
                    cat3_scr[m * B:(m + 1) * B].reshape(B, D_CAT)

    @pl.when((step >= LAG) & (step < FC2_START))
    def _fc1():
        c = step - LAG
        nb = jax.lax.rem(c, NBUF)
        w_copy(c, nb).wait()
        blk = jnp.dot(cat2_scr[...], w_buf[nb], preferred_element_type=F32)
        h_scr[:, pl.ds(c * FC1_BLK, FC1_BLK)] = \
            jnp.maximum(blk + fc1b_ref[...], 0.0)

    @pl.when((step >= LAG) & (step < FC1_CHUNKS))
    def _kick_next():
        w_copy(step, jax.lax.rem(step, NBUF)).start()

    @pl.when(step >= FC2_START)
    def _fc2():
        blk = jnp.dot(h_scr[...], fc2w_ref[...], preferred_element_type=F32)
        out_ref[...] = jnp.maximum(blk + fc2b_ref[...], 0.0)


def kernel(A_on, P_on, N_on, A_in, P_in, N_in, A_prox, P_prox, N_prox,
           A_X, P_X, N_X,
           W_on_0, b_on_0, W_on_1, b_on_1,
           W_in_0, b_in_0, W_in_1, b_in_1,
           W_prox_0, b_prox_0, W_prox_1, b_prox_1,
           fc1_W, fc1_b, fc2_W, fc2_b):
    r = lambda b: b.reshape(1, -1)
    const = lambda arr: pl.BlockSpec(arr.shape, lambda i: (0,) * arr.ndim)
    hbm = lambda: pl.BlockSpec(memory_space=pltpu.MemorySpace.HBM)
    data_inputs = (A_X, P_X, N_X, A_on, P_on, N_on, A_in, P_in, N_in,
                   A_prox, P_prox, N_prox)
    weight_inputs = (W_on_0, r(b_on_0), W_on_1, r(b_on_1),
                     W_in_0, r(b_in_0), W_in_1, r(b_in_1),
                     W_prox_0, r(b_prox_0), W_prox_1, r(b_prox_1))
    in_specs = [hbm() for _ in data_inputs]
    in_specs += [const(a) for a in weight_inputs]
    in_specs += [
        hbm(),
        pl.BlockSpec((1, FC1_BLK),
                     lambda i: (0, jnp.clip(i - LAG, 0, FC1_CHUNKS - 1))),
        pl.BlockSpec((D_H, FC2_BLK),
                     lambda i: (0, jnp.maximum(i - FC2_START, 0))),
        pl.BlockSpec((1, FC2_BLK),
                     lambda i: (0, jnp.maximum(i - FC2_START, 0))),
    ]
    out = pl.pallas_call(
        _fused_kernel,
        grid=(NSTEPS,),
        in_specs=in_specs,
        out_specs=pl.BlockSpec((NB, FC2_BLK),
                               lambda i: (0, jnp.maximum(i - FC2_START, 0))),
        out_shape=jax.ShapeDtypeStruct((NB, D_OUT), F32),
        scratch_shapes=[
            pltpu.VMEM((NB, N, 192), F32),
            pltpu.VMEM((NB, D_CAT), F32),
            pltpu.VMEM((NB, D_H), F32),
            pltpu.VMEM((NBUF, D_CAT, FC1_BLK), F32),
            pltpu.VMEM((3, B, N, 256), F32),
            pltpu.VMEM((9, B, N, N), F32),
            pltpu.SemaphoreType.DMA((NBUF,)),
            pltpu.SemaphoreType.DMA((3,)),
            pltpu.SemaphoreType.DMA((9,)),
        ],
    )(*data_inputs, *weight_inputs, fc1_W, r(fc1_b), fc2_W, r(fc2_b))
    return (out[0:B], out[B:2 * B], out[2 * B:3 * B])
